# Initial kernel scaffold; baseline (speedup 1.0000x reference)
import jax
import jax.numpy as jnp
from jax.experimental import pallas as pl


def kernel(x, edge_index, edge_label_index, W_pre, b_pre, W_g0, b_g0, W_g1, b_g1, W_post, b_post):
    raise NotImplementedError("write your pallas kernel here")


# speedup vs baseline: 14.1355x; 14.1355x over previous
import functools

import jax
import jax.numpy as jnp
from jax import lax
from jax.experimental import pallas as pl
from jax.experimental.pallas import tpu as pltpu
from jax.experimental.pallas import tpu_sc as plsc

N = 10000
E = 320000
EL = 20000
D = 128
H = 128

NC = 2
NS = 16
NW = NC * NS
NCH = E // 128
CH0 = NCH // NW
CHX = NCH - CH0 * NW
NPAD = 10240
RPT = NPAD // NS

NCH_FULL = EL // 128
REM = EL - NCH_FULL * 128

_MESH = plsc.VectorSubcoreMesh(
    core_axis_name="c", subcore_axis_name="s", num_cores=NC, num_subcores=NS)


def _worker_id():
    return lax.axis_index("c") * NS + lax.axis_index("s")


def _chunk_range(wid):
    n = CH0 + (wid < CHX).astype(jnp.int32)
    start = wid * CH0 + jnp.minimum(wid, CHX)
    return start, n


@functools.partial(
    pl.kernel,
    out_type=jax.ShapeDtypeStruct((NC * NPAD,), jnp.float32),
    mesh=_MESH,
    scratch_types=[
        pltpu.VMEM((128,), jnp.int32),
        pltpu.VMEM((128,), jnp.float32),
        pltpu.VMEM_SHARED((NPAD,), jnp.float32),
    ],
)
def _sc_degree(dst_hbm, ones_hbm, zeros1_hbm, deg_hbm, didx, ones_v, dacc):
    c = lax.axis_index("c")
    s = lax.axis_index("s")
    wid = _worker_id()
    pltpu.sync_copy(ones_hbm, ones_v)
    pltpu.sync_copy(zeros1_hbm.at[pl.ds(s * RPT, RPT)],
                    dacc.at[pl.ds(s * RPT, RPT)])
    plsc.subcore_barrier()

    start, n = _chunk_range(wid)

    @pl.loop(0, n)
    def _deg(k):
        ci = start + k
        pltpu.sync_copy(dst_hbm.at[pl.ds(ci * 128, 128)], didx)
        pltpu.sync_copy(ones_v, dacc.at[didx], add=True)

    plsc.subcore_barrier()
    pltpu.sync_copy(dacc.at[pl.ds(s * RPT, RPT)],
                    deg_hbm.at[pl.ds(c * NPAD + s * RPT, RPT)])


def _sc_agg_body(table_hbm, src_hbm, dst_hbm, zeros2_hbm, out_hbm,
                 sidx, didx, rows, acc, gsem):
    c = lax.axis_index("c")
    s = lax.axis_index("s")
    wid = _worker_id()

    pltpu.sync_copy(zeros2_hbm.at[pl.ds(s * RPT, RPT)],
                    acc.at[pl.ds(s * RPT, RPT)])
    plsc.subcore_barrier()

    start, n = _chunk_range(wid)

    @pl.loop(0, n)
    def _edges(k):
        ci = start + k
        pltpu.sync_copy(src_hbm.at[pl.ds(ci * 128, 128)], sidx)
        pltpu.sync_copy(dst_hbm.at[pl.ds(ci * 128, 128)], didx)
        pltpu.async_copy(table_hbm.at[sidx], rows, gsem).wait()
        pltpu.sync_copy(rows, acc.at[didx], add=True)

    plsc.subcore_barrier()
    pltpu.sync_copy(acc.at[pl.ds(s * RPT, RPT)],
                    out_hbm.at[c, pl.ds(s * RPT, RPT)])


_sc_aggregate = functools.partial(
    pl.kernel,
    out_type=jax.ShapeDtypeStruct((NC, NPAD, H), jnp.float32),
    mesh=_MESH,
    scratch_types=[
        pltpu.VMEM((128,), jnp.int32),
        pltpu.VMEM((128,), jnp.int32),
        pltpu.VMEM((128, H), jnp.float32),
        pltpu.VMEM_SHARED((NPAD, H), jnp.float32),
        pltpu.SemaphoreType.DMA,
    ],
)(_sc_agg_body)


@functools.partial(
    pl.kernel,
    out_type=[
        jax.ShapeDtypeStruct((EL, H), jnp.float32),
        jax.ShapeDtypeStruct((EL, H), jnp.float32),
    ],
    mesh=_MESH,
    scratch_types=[
        pltpu.VMEM((128,), jnp.int32),
        pltpu.VMEM((128,), jnp.int32),
        pltpu.VMEM((128, H), jnp.float32),
        pltpu.VMEM((128, H), jnp.float32),
        pltpu.SemaphoreType.DMA,
        pltpu.SemaphoreType.DMA,
    ],
)
def _sc_decode(emb_hbm, eli0_hbm, eli1_hbm, hs_hbm, hd_hbm,
               i0, i1, r0, r1, sem0, sem1):
    wid = _worker_id()

    for k in range(NCH_FULL // NW + 1):
        ci = wid + NW * k

        @pl.when(ci < NCH_FULL)
        def _full():
            pltpu.sync_copy(eli0_hbm.at[pl.ds(ci * 128, 128)], i0)
            pltpu.sync_copy(eli1_hbm.at[pl.ds(ci * 128, 128)], i1)
            pltpu.async_copy(emb_hbm.at[i0], r0, sem0).wait()
            pltpu.async_copy(emb_hbm.at[i1], r1, sem1).wait()
            pltpu.sync_copy(r0, hs_hbm.at[pl.ds(ci * 128, 128)])
            pltpu.sync_copy(r1, hd_hbm.at[pl.ds(ci * 128, 128)])

        @pl.when(ci == NCH_FULL)
        def _rem():
            pltpu.sync_copy(eli0_hbm.at[pl.ds(NCH_FULL * 128 - 96, 128)], i0)
            pltpu.sync_copy(eli1_hbm.at[pl.ds(NCH_FULL * 128 - 96, 128)], i1)
            pltpu.async_copy(emb_hbm.at[i0], r0, sem0).wait()
            pltpu.async_copy(emb_hbm.at[i1], r1, sem1).wait()
            pltpu.sync_copy(r0.at[pl.ds(128 - REM, REM)],
                            hs_hbm.at[pl.ds(NCH_FULL * 128, REM)])
            pltpu.sync_copy(r1.at[pl.ds(128 - REM, REM)],
                            hd_hbm.at[pl.ds(NCH_FULL * 128, REM)])


BR = 512
GR = NPAD // BR
GS = (EL + BR - 1) // BR


def _t1_body(x_ref, wpre_ref, bpre_ref, wg0_ref, dinv_ref, hws0_ref):
    h = jnp.maximum(
        jnp.dot(x_ref[...], wpre_ref[...], preferred_element_type=jnp.float32)
        + bpre_ref[...], 0.0)
    hws0_ref[...] = jnp.dot(
        h, wg0_ref[...], preferred_element_type=jnp.float32) * dinv_ref[...]


def _tc_pre(x, W_pre, b_pre2, W_g0, dinv_col):
    return pl.pallas_call(
        _t1_body,
        grid=(GR,),
        in_specs=[
            pl.BlockSpec((BR, D), lambda i: (i, 0)),
            pl.BlockSpec((D, H), lambda i: (0, 0)),
            pl.BlockSpec((1, H), lambda i: (0, 0)),
            pl.BlockSpec((H, H), lambda i: (0, 0)),
            pl.BlockSpec((BR, 1), lambda i: (i, 0)),
        ],
        out_specs=pl.BlockSpec((BR, H), lambda i: (i, 0)),
        out_shape=jax.ShapeDtypeStruct((N, H), jnp.float32),
    )(x, W_pre, b_pre2, W_g0, dinv_col)


def _t2_body(agg_ref, hws0_ref, dinv_ref, bg0_ref, wg1_ref, emb0_ref, hws1_ref):
    a = agg_ref[0] + agg_ref[1]
    pre = dinv_ref[...] * (a + hws0_ref[...]) + bg0_ref[...]
    e0 = jnp.maximum(pre, 0.0)
    emb0_ref[...] = e0
    hws1_ref[...] = jnp.dot(
        e0, wg1_ref[...], preferred_element_type=jnp.float32) * dinv_ref[...]


def _tc_mid(agg0, hws0, dinv_col, b_g02, W_g1):
    return pl.pallas_call(
        _t2_body,
        grid=(GR,),
        in_specs=[
            pl.BlockSpec((NC, BR, H), lambda i: (0, i, 0)),
            pl.BlockSpec((BR, H), lambda i: (i, 0)),
            pl.BlockSpec((BR, 1), lambda i: (i, 0)),
            pl.BlockSpec((1, H), lambda i: (0, 0)),
            pl.BlockSpec((H, H), lambda i: (0, 0)),
        ],
        out_specs=[
            pl.BlockSpec((BR, H), lambda i: (i, 0)),
            pl.BlockSpec((BR, H), lambda i: (i, 0)),
        ],
        out_shape=[
            jax.ShapeDtypeStruct((N, H), jnp.float32),
            jax.ShapeDtypeStruct((N, H), jnp.float32),
        ],
    )(agg0, hws0, dinv_col, b_g02, W_g1)


def _t3_body(agg_ref, hws1_ref, dinv_ref, bg1_ref, emb1_ref):
    a = agg_ref[0] + agg_ref[1]
    pre = dinv_ref[...] * (a + hws1_ref[...]) + bg1_ref[...]
    emb1_ref[...] = jnp.maximum(pre, 0.0)


def _tc_post(agg1, hws1, dinv_col, b_g12):
    return pl.pallas_call(
        _t3_body,
        grid=(GR,),
        in_specs=[
            pl.BlockSpec((NC, BR, H), lambda i: (0, i, 0)),
            pl.BlockSpec((BR, H), lambda i: (i, 0)),
            pl.BlockSpec((BR, 1), lambda i: (i, 0)),
            pl.BlockSpec((1, H), lambda i: (0, 0)),
        ],
        out_specs=pl.BlockSpec((BR, H), lambda i: (i, 0)),
        out_shape=jax.ShapeDtypeStruct((N, H), jnp.float32),
    )(agg1, hws1, dinv_col, b_g12)


def _t4_body(hs_ref, hd_ref, wpost_ref, bpost_ref, out_ref):
    had = hs_ref[...] * hd_ref[...]
    logits = jnp.dot(
        had, wpost_ref[...], preferred_element_type=jnp.float32) + bpost_ref[...]
    out_ref[...] = jnp.sum(logits, axis=1, keepdims=True)


def _tc_score(hs, hd, W_post, b_post2):
    return pl.pallas_call(
        _t4_body,
        grid=(GS,),
        in_specs=[
            pl.BlockSpec((BR, H), lambda i: (i, 0)),
            pl.BlockSpec((BR, H), lambda i: (i, 0)),
            pl.BlockSpec((H, 2), lambda i: (0, 0)),
            pl.BlockSpec((1, 2), lambda i: (0, 0)),
        ],
        out_specs=pl.BlockSpec((BR, 1), lambda i: (i, 0)),
        out_shape=jax.ShapeDtypeStruct((EL, 1), jnp.float32),
    )(hs, hd, W_post, b_post2)


def kernel(x, edge_index, edge_label_index,
           W_pre, b_pre, W_g0, b_g0, W_g1, b_g1, W_post, b_post):
    src = edge_index[0]
    dst = edge_index[1]
    eli0 = edge_label_index[0]
    eli1 = edge_label_index[1]

    ones128 = jnp.ones((128,), jnp.float32)
    zeros1 = jnp.zeros((NPAD,), jnp.float32)
    zeros2 = jnp.zeros((NPAD, H), jnp.float32)

    deg = _sc_degree(dst, ones128, zeros1)
    dtot = deg[:NPAD] + deg[NPAD:] + 1.0
    dinv_col = lax.rsqrt(dtot)[:, None]

    hws0 = _tc_pre(x, W_pre, b_pre.reshape(1, H), W_g0, dinv_col)
    agg0 = _sc_aggregate(hws0, src, dst, zeros2)
    emb0, hws1 = _tc_mid(agg0, hws0, dinv_col, b_g0.reshape(1, H), W_g1)
    agg1 = _sc_aggregate(hws1, src, dst, zeros2)
    emb1 = _tc_post(agg1, hws1, dinv_col, b_g1.reshape(1, H))

    hs, hd = _sc_decode(emb1, eli0, eli1)
    scores2 = _tc_score(hs, hd, W_post, b_post.reshape(1, 2))
    return scores2.reshape(EL), emb0, emb1


# speedup vs baseline: 21.6921x; 1.5346x over previous
import functools

import jax
import jax.numpy as jnp
from jax import lax
from jax.experimental import pallas as pl
from jax.experimental.pallas import tpu as pltpu
from jax.experimental.pallas import tpu_sc as plsc

N = 10000
E = 320000
EL = 20000
D = 128
H = 128

NC = 2
NS = 16
NW = NC * NS
NCH = E // 128
CH0 = NCH // NW
CHX = NCH - CH0 * NW
NPAD = 10240
RPT = NPAD // NS

NCH_FULL = EL // 128
REM = EL - NCH_FULL * 128

_MESH = plsc.VectorSubcoreMesh(
    core_axis_name="c", subcore_axis_name="s", num_cores=NC, num_subcores=NS)


def _worker_id():
    return lax.axis_index("c") * NS + lax.axis_index("s")


def _chunk_range(wid):
    n = CH0 + (wid < CHX).astype(jnp.int32)
    start = wid * CH0 + jnp.minimum(wid, CHX)
    return start, n


@functools.partial(
    pl.kernel,
    out_type=jax.ShapeDtypeStruct((NC * NPAD,), jnp.float32),
    mesh=_MESH,
    scratch_types=[
        pltpu.VMEM((128,), jnp.int32),
        pltpu.VMEM((128,), jnp.float32),
        pltpu.VMEM_SHARED((NPAD,), jnp.float32),
    ],
)
def _sc_degree(dst_hbm, ones_hbm, zeros1_hbm, deg_hbm, didx, ones_v, dacc):
    c = lax.axis_index("c")
    s = lax.axis_index("s")
    wid = _worker_id()
    pltpu.sync_copy(ones_hbm, ones_v)
    pltpu.sync_copy(zeros1_hbm.at[pl.ds(s * RPT, RPT)],
                    dacc.at[pl.ds(s * RPT, RPT)])
    plsc.subcore_barrier()

    start, n = _chunk_range(wid)

    @pl.loop(0, n)
    def _deg(k):
        ci = start + k
        pltpu.sync_copy(dst_hbm.at[pl.ds(ci * 128, 128)], didx)
        pltpu.sync_copy(ones_v, dacc.at[didx], add=True)

    plsc.subcore_barrier()
    pltpu.sync_copy(dacc.at[pl.ds(s * RPT, RPT)],
                    deg_hbm.at[pl.ds(c * NPAD + s * RPT, RPT)])


def _sc_agg_body(table_hbm, src_hbm, dst_hbm, zeros2_hbm, out_hbm,
                 sidx_all, didx_a, didx_b, rows_a, rows_b, acc,
                 sem_a, sem_b):
    c = lax.axis_index("c")
    s = lax.axis_index("s")
    wid = _worker_id()

    pltpu.sync_copy(zeros2_hbm.at[pl.ds(s * RPT, RPT)],
                    acc.at[pl.ds(s * RPT, RPT)])

    start, _ = _chunk_range(wid)
    pltpu.sync_copy(src_hbm.at[pl.ds(start * 128, CH0 * 128)], sidx_all)
    plsc.subcore_barrier()

    def _fetch_didx(k, buf):
        pltpu.sync_copy(dst_hbm.at[pl.ds((start + k) * 128, 128)], buf)

    def _start_gather(k, rows, sem):
        pltpu.async_copy(
            table_hbm.at[sidx_all.at[pl.ds(k * 128, 128)]], rows, sem)

    _fetch_didx(0, didx_a)
    _start_gather(0, rows_a, sem_a)
    _fetch_didx(1, didx_b)
    _start_gather(1, rows_b, sem_b)

    @pl.loop(0, CH0 // 2)
    def _pairs(j):
        k0 = 2 * j
        pltpu.make_async_copy(table_hbm.at[pl.ds(0, 128)], rows_a, sem_a).wait()
        pltpu.sync_copy(rows_a, acc.at[didx_a], add=True)

        @pl.when(k0 + 2 < CH0)
        def _next_a():
            _fetch_didx(k0 + 2, didx_a)
            _start_gather(k0 + 2, rows_a, sem_a)

        pltpu.make_async_copy(table_hbm.at[pl.ds(0, 128)], rows_b, sem_b).wait()
        pltpu.sync_copy(rows_b, acc.at[didx_b], add=True)

        @pl.when(k0 + 3 < CH0)
        def _next_b():
            _fetch_didx(k0 + 3, didx_b)
            _start_gather(k0 + 3, rows_b, sem_b)

    @pl.when(wid < CHX)
    def _extra():
        ce = NCH - CHX + wid
        pltpu.sync_copy(dst_hbm.at[pl.ds(ce * 128, 128)], didx_a)
        pltpu.sync_copy(src_hbm.at[pl.ds(ce * 128, 128)], didx_b)
        pltpu.async_copy(table_hbm.at[didx_b], rows_a, sem_a).wait()
        pltpu.sync_copy(rows_a, acc.at[didx_a], add=True)

    plsc.subcore_barrier()
    pltpu.sync_copy(acc.at[pl.ds(s * RPT, RPT)],
                    out_hbm.at[c, pl.ds(s * RPT, RPT)])


_sc_aggregate = functools.partial(
    pl.kernel,
    out_type=jax.ShapeDtypeStruct((NC, NPAD, H), jnp.float32),
    mesh=_MESH,
    scratch_types=[
        pltpu.VMEM((CH0 * 128,), jnp.int32),
        pltpu.VMEM((128,), jnp.int32),
        pltpu.VMEM((128,), jnp.int32),
        pltpu.VMEM((128, H), jnp.float32),
        pltpu.VMEM((128, H), jnp.float32),
        pltpu.VMEM_SHARED((NPAD, H), jnp.float32),
        pltpu.SemaphoreType.DMA,
        pltpu.SemaphoreType.DMA,
    ],
)(_sc_agg_body)


@functools.partial(
    pl.kernel,
    out_type=[
        jax.ShapeDtypeStruct((EL, H), jnp.float32),
        jax.ShapeDtypeStruct((EL, H), jnp.float32),
    ],
    mesh=_MESH,
    scratch_types=[
        pltpu.VMEM((128,), jnp.int32),
        pltpu.VMEM((128,), jnp.int32),
        pltpu.VMEM((128, H), jnp.float32),
        pltpu.VMEM((128, H), jnp.float32),
        pltpu.SemaphoreType.DMA,
        pltpu.SemaphoreType.DMA,
    ],
)
def _sc_decode(emb_hbm, eli0_hbm, eli1_hbm, hs_hbm, hd_hbm,
               i0, i1, r0, r1, sem0, sem1):
    wid = _worker_id()

    for k in range(NCH_FULL // NW + 1):
        ci = wid + NW * k

        @pl.when(ci < NCH_FULL)
        def _full():
            pltpu.sync_copy(eli0_hbm.at[pl.ds(ci * 128, 128)], i0)
            pltpu.sync_copy(eli1_hbm.at[pl.ds(ci * 128, 128)], i1)
            pltpu.async_copy(emb_hbm.at[i0], r0, sem0).wait()
            pltpu.async_copy(emb_hbm.at[i1], r1, sem1).wait()
            pltpu.sync_copy(r0, hs_hbm.at[pl.ds(ci * 128, 128)])
            pltpu.sync_copy(r1, hd_hbm.at[pl.ds(ci * 128, 128)])

        @pl.when(ci == NCH_FULL)
        def _rem():
            pltpu.sync_copy(eli0_hbm.at[pl.ds(NCH_FULL * 128 - 96, 128)], i0)
            pltpu.sync_copy(eli1_hbm.at[pl.ds(NCH_FULL * 128 - 96, 128)], i1)
            pltpu.async_copy(emb_hbm.at[i0], r0, sem0).wait()
            pltpu.async_copy(emb_hbm.at[i1], r1, sem1).wait()
            pltpu.sync_copy(r0.at[pl.ds(128 - REM, REM)],
                            hs_hbm.at[pl.ds(NCH_FULL * 128, REM)])
            pltpu.sync_copy(r1.at[pl.ds(128 - REM, REM)],
                            hd_hbm.at[pl.ds(NCH_FULL * 128, REM)])


BR = 512
GR = NPAD // BR
GS = (EL + BR - 1) // BR


def _t1_body(x_ref, wpre_ref, bpre_ref, wg0_ref, hw0_ref):
    h = jnp.maximum(
        jnp.dot(x_ref[...], wpre_ref[...], preferred_element_type=jnp.float32)
        + bpre_ref[...], 0.0)
    hw0_ref[...] = jnp.dot(h, wg0_ref[...], preferred_element_type=jnp.float32)


def _tc_pre(x, W_pre, b_pre2, W_g0):
    return pl.pallas_call(
        _t1_body,
        grid=(GR,),
        in_specs=[
            pl.BlockSpec((BR, D), lambda i: (i, 0)),
            pl.BlockSpec((D, H), lambda i: (0, 0)),
            pl.BlockSpec((1, H), lambda i: (0, 0)),
            pl.BlockSpec((H, H), lambda i: (0, 0)),
        ],
        out_specs=pl.BlockSpec((BR, H), lambda i: (i, 0)),
        out_shape=jax.ShapeDtypeStruct((N, H), jnp.float32),
    )(x, W_pre, b_pre2, W_g0)


def _t1s_body(hw_ref, dinv_ref, hws_ref):
    hws_ref[...] = hw_ref[...] * dinv_ref[...]


def _tc_scale(hw0, dinv_col):
    return pl.pallas_call(
        _t1s_body,
        grid=(GR,),
        in_specs=[
            pl.BlockSpec((BR, H), lambda i: (i, 0)),
            pl.BlockSpec((BR, 1), lambda i: (i, 0)),
        ],
        out_specs=pl.BlockSpec((BR, H), lambda i: (i, 0)),
        out_shape=jax.ShapeDtypeStruct((N, H), jnp.float32),
    )(hw0, dinv_col)


def _t2_body(agg_ref, hws0_ref, dinv_ref, bg0_ref, wg1_ref, emb0_ref, hws1_ref):
    a = agg_ref[0] + agg_ref[1]
    pre = dinv_ref[...] * (a + hws0_ref[...]) + bg0_ref[...]
    e0 = jnp.maximum(pre, 0.0)
    emb0_ref[...] = e0
    hws1_ref[...] = jnp.dot(
        e0, wg1_ref[...], preferred_element_type=jnp.float32) * dinv_ref[...]


def _tc_mid(agg0, hws0, dinv_col, b_g02, W_g1):
    return pl.pallas_call(
        _t2_body,
        grid=(GR,),
        in_specs=[
            pl.BlockSpec((NC, BR, H), lambda i: (0, i, 0)),
            pl.BlockSpec((BR, H), lambda i: (i, 0)),
            pl.BlockSpec((BR, 1), lambda i: (i, 0)),
            pl.BlockSpec((1, H), lambda i: (0, 0)),
            pl.BlockSpec((H, H), lambda i: (0, 0)),
        ],
        out_specs=[
            pl.BlockSpec((BR, H), lambda i: (i, 0)),
            pl.BlockSpec((BR, H), lambda i: (i, 0)),
        ],
        out_shape=[
            jax.ShapeDtypeStruct((N, H), jnp.float32),
            jax.ShapeDtypeStruct((N, H), jnp.float32),
        ],
    )(agg0, hws0, dinv_col, b_g02, W_g1)


def _t3_body(agg_ref, hws1_ref, dinv_ref, bg1_ref, emb1_ref):
    a = agg_ref[0] + agg_ref[1]
    pre = dinv_ref[...] * (a + hws1_ref[...]) + bg1_ref[...]
    emb1_ref[...] = jnp.maximum(pre, 0.0)


def _tc_post(agg1, hws1, dinv_col, b_g12):
    return pl.pallas_call(
        _t3_body,
        grid=(GR,),
        in_specs=[
            pl.BlockSpec((NC, BR, H), lambda i: (0, i, 0)),
            pl.BlockSpec((BR, H), lambda i: (i, 0)),
            pl.BlockSpec((BR, 1), lambda i: (i, 0)),
            pl.BlockSpec((1, H), lambda i: (0, 0)),
        ],
        out_specs=pl.BlockSpec((BR, H), lambda i: (i, 0)),
        out_shape=jax.ShapeDtypeStruct((N, H), jnp.float32),
    )(agg1, hws1, dinv_col, b_g12)


def _t4_body(hs_ref, hd_ref, wpost_ref, bpost_ref, out_ref):
    had = hs_ref[...] * hd_ref[...]
    logits = jnp.dot(
        had, wpost_ref[...], preferred_element_type=jnp.float32) + bpost_ref[...]
    out_ref[...] = jnp.sum(logits, axis=1, keepdims=True)


def _tc_score(hs, hd, W_post, b_post2):
    return pl.pallas_call(
        _t4_body,
        grid=(GS,),
        in_specs=[
            pl.BlockSpec((BR, H), lambda i: (i, 0)),
            pl.BlockSpec((BR, H), lambda i: (i, 0)),
            pl.BlockSpec((H, 2), lambda i: (0, 0)),
            pl.BlockSpec((1, 2), lambda i: (0, 0)),
        ],
        out_specs=pl.BlockSpec((BR, 1), lambda i: (i, 0)),
        out_shape=jax.ShapeDtypeStruct((EL, 1), jnp.float32),
    )(hs, hd, W_post, b_post2)


def kernel(x, edge_index, edge_label_index,
           W_pre, b_pre, W_g0, b_g0, W_g1, b_g1, W_post, b_post):
    src = edge_index[0]
    dst = edge_index[1]
    eli0 = edge_label_index[0]
    eli1 = edge_label_index[1]

    ones128 = jnp.ones((128,), jnp.float32)
    zeros1 = jnp.zeros((NPAD,), jnp.float32)
    zeros2 = jnp.zeros((NPAD, H), jnp.float32)

    deg = _sc_degree(dst, ones128, zeros1)
    hw0 = _tc_pre(x, W_pre, b_pre.reshape(1, H), W_g0)
    dtot = deg[:NPAD] + deg[NPAD:] + 1.0
    dinv_col = lax.rsqrt(dtot)[:, None]
    hws0 = _tc_scale(hw0, dinv_col)
    agg0 = _sc_aggregate(hws0, src, dst, zeros2)
    emb0, hws1 = _tc_mid(agg0, hws0, dinv_col, b_g0.reshape(1, H), W_g1)
    agg1 = _sc_aggregate(hws1, src, dst, zeros2)
    emb1 = _tc_post(agg1, hws1, dinv_col, b_g1.reshape(1, H))

    hs, hd = _sc_decode(emb1, eli0, eli1)
    scores2 = _tc_score(hs, hd, W_post, b_post.reshape(1, 2))
    return scores2.reshape(EL), emb0, emb1
